# baseline (speedup 1.0000x reference)
import functools

import jax
import jax.numpy as jnp
from jax import lax
from jax.experimental import pallas as pl
from jax.experimental.pallas import tpu as pltpu
from jax.experimental.pallas import tpu_sc as plsc

B = 16384
P = 4
T = 81
C = 39
L = 16
NC = 2
NS = 16
NW = NC * NS
RW = B // NW
RTC = 256
GRID = B // RTC


def _tc_body(p_ref, x_ref, y_ref):
    xb = x_ref[...]
    pb = p_ref[...]
    xx = jnp.sum(jnp.sum(xb * xb, axis=2), axis=1)
    cols = []
    for p in range(P):
        pbp = pb[p]
        xp = jnp.sum(jnp.sum(xb * pbp, axis=2), axis=1)
        pp = jnp.sum(pbp * pbp)
        cols.append(xx - 2.0 * xp + pp)
    y_ref[...] = jnp.stack(cols, axis=1)


@jax.jit
def _tc_call(x, prototypes):
    return pl.pallas_call(
        _tc_body,
        grid=(GRID,),
        in_specs=[
            pl.BlockSpec((P, T, C), lambda i: (0, 0, 0)),
            pl.BlockSpec((RTC, T, C), lambda i: (i, 0, 0)),
        ],
        out_specs=pl.BlockSpec((RTC, P), lambda i: (i, 0)),
        out_shape=jax.ShapeDtypeStruct((B, P), jnp.float32),
    )(prototypes, x)


def _sc_body(y_hbm, a_hbm, ybuf, abuf, sem):
    wid = lax.axis_index("s") * NC + lax.axis_index("c")
    pltpu.async_copy(y_hbm.at[pl.ds(wid * RW, RW)], ybuf, sem)
    lane = lax.iota(jnp.int32, L)
    pltpu.make_async_copy(y_hbm.at[pl.ds(wid * RW, RW)], ybuf, sem).wait()

    def group_body(i, carry):
        rowidx = i * L + lane
        ys = [plsc.load_gather(ybuf, [rowidx, jnp.full((L,), p, jnp.int32)])
              for p in range(P)]
        m = ys[0]
        am = jnp.zeros((L,), jnp.int32)
        for p in range(1, P):
            lt = ys[p] < m
            am = jnp.where(lt, p, am)
            m = jnp.where(lt, ys[p], m)
        plsc.store_scatter(abuf, [i * L + lane], am)
        return carry

    lax.fori_loop(0, RW // L, group_body, 0)
    pltpu.sync_copy(abuf, a_hbm.at[pl.ds(wid * RW, RW)])


@functools.lru_cache(maxsize=1)
def _build_sc_call():
    mesh = plsc.VectorSubcoreMesh(core_axis_name="c", subcore_axis_name="s",
                                  num_cores=NC, num_subcores=NS)
    return functools.partial(
        pl.kernel,
        out_type=jax.ShapeDtypeStruct((B,), jnp.int32),
        mesh=mesh,
        scratch_types=[
            pltpu.VMEM((RW, P), jnp.float32),
            pltpu.VMEM((RW,), jnp.int32),
            pltpu.SemaphoreType.DMA,
        ],
        compiler_params=pltpu.CompilerParams(needs_layout_passes=False,
                                             use_tc_tiling_on_sc=False),
    )(_sc_body)


def kernel(x, prototypes):
    y = _tc_call(x, prototypes)
    am = _build_sc_call()(y)
    return (y, am)


# speedup vs baseline: 2.5477x; 2.5477x over previous
import functools

import jax
import jax.numpy as jnp
from jax import lax
from jax.experimental import pallas as pl
from jax.experimental.pallas import tpu as pltpu
from jax.experimental.pallas import tpu_sc as plsc

B = 16384
P = 4
T = 81
C = 39
D = T * C
L = 16
NC = 2
NS = 16
NW = NC * NS
RW = B // NW
RTC = 256
GRID = B // RTC


def _tc_body(w_ref, x_ref, y_ref):
    xb = x_ref[...]
    wb = w_ref[...]
    dot = jnp.dot(xb, wb, preferred_element_type=jnp.float32,
                  precision=lax.Precision.HIGHEST)
    xx = jnp.sum(xb * xb, axis=1)
    pp = jnp.sum(wb * wb, axis=0)
    y_ref[...] = xx[:, None] - 2.0 * dot + pp[None, :]


@jax.jit
def _tc_call(x2, wT):
    return pl.pallas_call(
        _tc_body,
        grid=(GRID,),
        in_specs=[
            pl.BlockSpec((D, P), lambda i: (0, 0)),
            pl.BlockSpec((RTC, D), lambda i: (i, 0)),
        ],
        out_specs=pl.BlockSpec((RTC, P), lambda i: (i, 0)),
        out_shape=jax.ShapeDtypeStruct((B, P), jnp.float32),
    )(wT, x2)


def _sc_body(y_hbm, a_hbm, ybuf, abuf, sem):
    wid = lax.axis_index("s") * NC + lax.axis_index("c")
    pltpu.async_copy(y_hbm.at[pl.ds(wid * RW, RW)], ybuf, sem)
    lane = lax.iota(jnp.int32, L)
    pltpu.make_async_copy(y_hbm.at[pl.ds(wid * RW, RW)], ybuf, sem).wait()

    def group_body(i, carry):
        rowidx = i * L + lane
        ys = [plsc.load_gather(ybuf, [rowidx, jnp.full((L,), p, jnp.int32)])
              for p in range(P)]
        m = ys[0]
        am = jnp.zeros((L,), jnp.int32)
        for p in range(1, P):
            lt = ys[p] < m
            am = jnp.where(lt, p, am)
            m = jnp.where(lt, ys[p], m)
        plsc.store_scatter(abuf, [i * L + lane], am)
        return carry

    lax.fori_loop(0, RW // L, group_body, 0)
    pltpu.sync_copy(abuf, a_hbm.at[pl.ds(wid * RW, RW)])


@functools.lru_cache(maxsize=1)
def _build_sc_call():
    mesh = plsc.VectorSubcoreMesh(core_axis_name="c", subcore_axis_name="s",
                                  num_cores=NC, num_subcores=NS)
    return functools.partial(
        pl.kernel,
        out_type=jax.ShapeDtypeStruct((B,), jnp.int32),
        mesh=mesh,
        scratch_types=[
            pltpu.VMEM((RW, P), jnp.float32),
            pltpu.VMEM((RW,), jnp.int32),
            pltpu.SemaphoreType.DMA,
        ],
        compiler_params=pltpu.CompilerParams(needs_layout_passes=False,
                                             use_tc_tiling_on_sc=False),
    )(_sc_body)


def kernel(x, prototypes):
    x2 = x.reshape(B, D)
    wT = prototypes.reshape(P, D).T
    y = _tc_call(x2, wT)
    am = _build_sc_call()(y)
    return (y, am)


# speedup vs baseline: 12.4887x; 4.9020x over previous
import functools

import jax
import jax.numpy as jnp
from jax import lax
from jax.experimental import pallas as pl
from jax.experimental.pallas import tpu as pltpu
from jax.experimental.pallas import tpu_sc as plsc

B = 16384
P = 4
T = 81
C = 39
L = 16
NC = 2
NS = 16
NW = NC * NS
RW = B // NW
RTC = 512
GRID = B // RTC


def _tc_body(p_ref, x_ref, y_ref):
    xb = x_ref[...]
    cols = []
    for p in range(P):
        d = xb - p_ref[:, :, p][:, :, None]
        cols.append(jnp.sum(jnp.sum(d * d, axis=0), axis=0))
    y_ref[...] = jnp.stack(cols, axis=0)


@jax.jit
def _tc_call(xt, pt):
    return pl.pallas_call(
        _tc_body,
        grid=(GRID,),
        in_specs=[
            pl.BlockSpec((T, C, P), lambda i: (0, 0, 0)),
            pl.BlockSpec((T, C, RTC), lambda i: (0, 0, i)),
        ],
        out_specs=pl.BlockSpec((P, RTC), lambda i: (0, i)),
        out_shape=jax.ShapeDtypeStruct((P, B), jnp.float32),
    )(pt, xt)


def _sc_body(yt_hbm, a_hbm, ybuf, abuf, sem):
    wid = lax.axis_index("s") * NC + lax.axis_index("c")
    pltpu.async_copy(yt_hbm.at[:, pl.ds(wid * RW, RW)], ybuf, sem)
    lane = lax.iota(jnp.int32, L)
    pltpu.make_async_copy(yt_hbm.at[:, pl.ds(wid * RW, RW)], ybuf,
                          sem).wait()

    def group_body(i, carry):
        ys = [ybuf[p, pl.ds(i * L, L)] for p in range(P)]
        m = ys[0]
        am = jnp.zeros((L,), jnp.int32)
        for p in range(1, P):
            lt = ys[p] < m
            am = jnp.where(lt, p, am)
            m = jnp.where(lt, ys[p], m)
        plsc.store_scatter(abuf, [i * L + lane], am)
        return carry

    lax.fori_loop(0, RW // L, group_body, 0)
    pltpu.sync_copy(abuf, a_hbm.at[pl.ds(wid * RW, RW)])


@functools.lru_cache(maxsize=1)
def _build_sc_call():
    mesh = plsc.VectorSubcoreMesh(core_axis_name="c", subcore_axis_name="s",
                                  num_cores=NC, num_subcores=NS)
    return functools.partial(
        pl.kernel,
        out_type=jax.ShapeDtypeStruct((B,), jnp.int32),
        mesh=mesh,
        scratch_types=[
            pltpu.VMEM((P, RW), jnp.float32),
            pltpu.VMEM((RW,), jnp.int32),
            pltpu.SemaphoreType.DMA,
        ],
        compiler_params=pltpu.CompilerParams(needs_layout_passes=False,
                                             use_tc_tiling_on_sc=False),
    )(_sc_body)


def kernel(x, prototypes):
    xt = x.transpose(1, 2, 0)
    pt = prototypes.transpose(1, 2, 0)
    yt = _tc_call(xt, pt)
    am = _build_sc_call()(yt)
    y = yt.T
    return (y, am)
